# Initial kernel scaffold; baseline (speedup 1.0000x reference)
import functools

import jax
import jax.numpy as jnp
from jax import lax
from jax.experimental import pallas as pl
from jax.experimental.pallas import tpu as pltpu
from jax.experimental.pallas import tpu_sc as plsc

M = 320000
C_S = 128
NUM_SEGMENTS = 10000

NC = 2
NS = 16
NW = NC * NS

ROWS_PER_W = M // NW
CHUNK = 125
CHUNKS_PER_W = ROWS_PER_W // CHUNK
STRIPE = NUM_SEGMENTS // NS

_sc_mesh = plsc.VectorSubcoreMesh(core_axis_name="c", subcore_axis_name="s")


@functools.partial(
    pl.kernel,
    out_type=[
        jax.ShapeDtypeStruct((NC * NUM_SEGMENTS, C_S), jnp.float32),
        jax.ShapeDtypeStruct((NC * NUM_SEGMENTS, 16), jnp.float32),
    ],
    mesh=_sc_mesh,
    scratch_types=[
        pltpu.VMEM((CHUNK, C_S), jnp.float32),
        pltpu.VMEM((CHUNKS_PER_W, CHUNK), jnp.int32),
        pltpu.VMEM((CHUNK, 16), jnp.float32),
        pltpu.VMEM_SHARED((NUM_SEGMENTS, C_S), jnp.float32),
        pltpu.VMEM_SHARED((NUM_SEGMENTS, 16), jnp.float32),
    ],
)
def _sc_segment_sum(s_hbm, idx_hbm, z128_hbm, z16_hbm, sums_hbm, cnts_hbm,
                    rows_v, idx_v, ones_v, acc, cacc):
    c = lax.axis_index("c")
    sid = lax.axis_index("s")
    wid = sid * NC + c
    base_row = wid * ROWS_PER_W

    def _fill(i, carry):
        ones_v[i] = jnp.ones((16,), jnp.float32)
        return carry

    lax.fori_loop(0, CHUNK, _fill, 0)

    pltpu.sync_copy(z128_hbm.at[pl.ds(sid * STRIPE, STRIPE)],
                    acc.at[pl.ds(sid * STRIPE, STRIPE)])
    pltpu.sync_copy(z16_hbm.at[pl.ds(sid * STRIPE, STRIPE)],
                    cacc.at[pl.ds(sid * STRIPE, STRIPE)])

    pltpu.sync_copy(idx_hbm.at[pl.ds(wid * CHUNKS_PER_W, CHUNKS_PER_W)], idx_v)

    plsc.subcore_barrier()

    def _chunk(j, carry):
        pltpu.sync_copy(s_hbm.at[pl.ds(base_row + j * CHUNK, CHUNK)], rows_v)
        pltpu.sync_copy(rows_v, acc.at[idx_v.at[j]], add=True)
        pltpu.sync_copy(ones_v, cacc.at[idx_v.at[j]], add=True)
        return carry

    lax.fori_loop(0, CHUNKS_PER_W, _chunk, 0)

    plsc.subcore_barrier()

    pltpu.sync_copy(
        acc.at[pl.ds(sid * STRIPE, STRIPE)],
        sums_hbm.at[pl.ds(c * NUM_SEGMENTS + sid * STRIPE, STRIPE)])
    pltpu.sync_copy(
        cacc.at[pl.ds(sid * STRIPE, STRIPE)],
        cnts_hbm.at[pl.ds(c * NUM_SEGMENTS + sid * STRIPE, STRIPE)])


_B = 1000


def _tc_head(sums_ref, cnts_ref, w1_ref, b1_ref, w2_ref, b2_ref, out_ref):
    ssum = sums_ref[0] + sums_ref[1]
    cnt = cnts_ref[0, :, 0:1] + cnts_ref[1, :, 0:1]
    mean = ssum / jnp.maximum(cnt, 1.0)
    h = lax.dot_general(mean, w1_ref[...], (((1,), (1,)), ((), ())),
                        preferred_element_type=jnp.float32)
    h = jnp.maximum(h + b1_ref[...], 0.0)
    o = lax.dot_general(h, w2_ref[...], (((1,), (1,)), ((), ())),
                        preferred_element_type=jnp.float32)
    o = o + b2_ref[...]
    out_ref[...] = jnp.maximum(o, 0.0) + jnp.log1p(jnp.exp(-jnp.abs(o)))


def kernel(s, batch_vec, W1, b1, W2, b2):
    idx2 = jnp.asarray(batch_vec, jnp.int32).reshape(NW * CHUNKS_PER_W, CHUNK)
    z128 = jnp.zeros((NUM_SEGMENTS, C_S), jnp.float32)
    z16 = jnp.zeros((NUM_SEGMENTS, 16), jnp.float32)

    sums_flat, cnts_flat = _sc_segment_sum(s, idx2, z128, z16)
    sums2 = sums_flat.reshape(NC, NUM_SEGMENTS, C_S)
    cnts2 = cnts_flat.reshape(NC, NUM_SEGMENTS, 16)

    W2p = jnp.zeros((8, C_S), jnp.float32).at[:6].set(W2)
    b2p = jnp.zeros((1, 8), jnp.float32).at[0, :6].set(b2)
    b1r = b1.reshape(1, C_S)

    out8 = pl.pallas_call(
        _tc_head,
        grid=(NUM_SEGMENTS // _B,),
        in_specs=[
            pl.BlockSpec((NC, _B, C_S), lambda i: (0, i, 0)),
            pl.BlockSpec((NC, _B, 16), lambda i: (0, i, 0)),
            pl.BlockSpec((C_S, C_S), lambda i: (0, 0)),
            pl.BlockSpec((1, C_S), lambda i: (0, 0)),
            pl.BlockSpec((8, C_S), lambda i: (0, 0)),
            pl.BlockSpec((1, 8), lambda i: (0, 0)),
        ],
        out_specs=pl.BlockSpec((_B, 8), lambda i: (i, 0)),
        out_shape=jax.ShapeDtypeStruct((NUM_SEGMENTS, 8), jnp.float32),
    )(sums2, cnts2, W1, b1r, W2p, b2p)

    return out8[:, :6]


# speedup vs baseline: 2.7927x; 2.7927x over previous
import functools

import jax
import jax.numpy as jnp
from jax import lax
from jax.experimental import pallas as pl
from jax.experimental.pallas import tpu as pltpu
from jax.experimental.pallas import tpu_sc as plsc

M = 320000
C_S = 128
NUM_SEGMENTS = 10000

NC = 2
NS = 16

ROWS_PER_T = M // NS
CHUNK = 80
CHUNKS_PER_T = ROWS_PER_T // CHUNK
SEG_HALF = 5120
SEG_HPAD = 5248
STRIPE = SEG_HPAD // NS

def _sc_segment_sum_body(s_hbm, clidx_hbm, z_hbm, sums_hbm,
                         rows_v, idx_v, hop_v, acc):
    c = lax.axis_index("c")
    sid = lax.axis_index("s")

    pltpu.sync_copy(z_hbm, hop_v)
    pltpu.sync_copy(hop_v, acc.at[pl.ds(sid * STRIPE, STRIPE)])
    pltpu.sync_copy(clidx_hbm.at[c * NS + sid], idx_v)

    plsc.subcore_barrier()

    def _chunk(j, carry):
        pltpu.sync_copy(s_hbm.at[pl.ds(sid * ROWS_PER_T + j * CHUNK, CHUNK)],
                        rows_v)
        pltpu.sync_copy(rows_v, acc.at[idx_v.at[j]], add=True)
        return carry

    lax.fori_loop(0, CHUNKS_PER_T, _chunk, 0)

    plsc.subcore_barrier()

    pltpu.sync_copy(acc.at[pl.ds(sid * STRIPE, STRIPE)], hop_v)
    pltpu.sync_copy(hop_v, sums_hbm.at[pl.ds(c * SEG_HPAD + sid * STRIPE,
                                             STRIPE)])


@functools.lru_cache(maxsize=1)
def _sc_segment_sum():
    mesh = plsc.VectorSubcoreMesh(core_axis_name="c", subcore_axis_name="s",
                                  num_cores=NC, num_subcores=NS)
    return pl.kernel(
        _sc_segment_sum_body,
        out_type=jax.ShapeDtypeStruct((NC * SEG_HPAD, C_S), jnp.float32),
        mesh=mesh,
        scratch_types=[
            pltpu.VMEM((CHUNK, C_S), jnp.float32),
            pltpu.VMEM((CHUNKS_PER_T, CHUNK), jnp.int32),
            pltpu.VMEM((STRIPE, C_S), jnp.float32),
            pltpu.VMEM_SHARED((SEG_HPAD, C_S), jnp.float32),
        ],
    )


_IB = 1280
_NIB = M // _IB
_CROWS = 2 * SEG_HALF // 128


def _tc_counts(ids_ref, out_ref):
    i = pl.program_id(0)

    @pl.when(i == 0)
    def _():
        out_ref[...] = jnp.zeros((_CROWS, 128), jnp.float32)

    ids = ids_ref[0]
    lo = ids_ref[0, 0, 0]
    hi = ids_ref[0, _IB - 1, 0]
    r0 = lo // 128
    nwin = hi // 128 - r0 + 1
    col = lax.broadcasted_iota(jnp.int32, (1, 128), 1)

    def _win(w, carry):
        base = (r0 + w) * 128
        e = (ids == base + col).astype(jnp.float32)
        cw = jnp.sum(e, axis=0, keepdims=True)
        out_ref[pl.ds(r0 + w, 1), :] += cw
        return carry

    lax.fori_loop(0, nwin, _win, 0)


_B = 1024


def _tc_head(sums_ref, cnts_ref, w1_ref, b1_ref, w2_ref, b2_ref, out_ref):
    cnt = cnts_ref[...]
    mean = sums_ref[0] / jnp.maximum(cnt, 1.0)
    h = lax.dot_general(mean, w1_ref[...], (((1,), (1,)), ((), ())),
                        preferred_element_type=jnp.float32)
    h = jnp.maximum(h + b1_ref[...], 0.0)
    o = lax.dot_general(h, w2_ref[...], (((1,), (1,)), ((), ())),
                        preferred_element_type=jnp.float32)
    o = o + b2_ref[...]
    out_ref[...] = jnp.maximum(o, 0.0) + jnp.log1p(jnp.exp(-jnp.abs(o)))


def kernel(s, batch_vec, W1, b1, W2, b2):
    bv = jnp.asarray(batch_vec, jnp.int32)

    def _clamp(c):
        lo = c * SEG_HALF
        rel = bv - lo
        ok = (rel >= 0) & (rel < SEG_HALF)
        return jnp.where(ok, rel, SEG_HALF).reshape(NS, CHUNKS_PER_T, CHUNK)

    clidx = jnp.concatenate([_clamp(0), _clamp(1)], axis=0)
    z = jnp.zeros((STRIPE, C_S), jnp.float32)

    sums_flat = _sc_segment_sum()(s, clidx, z)
    sums3 = sums_flat.reshape(NC, SEG_HPAD, C_S)

    ids3 = bv.reshape(_NIB, _IB, 1)
    cnts2d = pl.pallas_call(
        _tc_counts,
        grid=(_NIB,),
        in_specs=[pl.BlockSpec((1, _IB, 1), lambda i: (i, 0, 0))],
        out_specs=pl.BlockSpec((_CROWS, 128), lambda i: (0, 0)),
        out_shape=jax.ShapeDtypeStruct((_CROWS, 128), jnp.float32),
    )(ids3)
    cnts = cnts2d.reshape(NC * SEG_HALF, 1)

    W2p = jnp.zeros((8, C_S), jnp.float32).at[:6].set(W2)
    b2p = jnp.zeros((1, 8), jnp.float32).at[0, :6].set(b2)
    b1r = b1.reshape(1, C_S)

    nb_half = SEG_HALF // _B
    out8 = pl.pallas_call(
        _tc_head,
        grid=(NC * nb_half,),
        in_specs=[
            pl.BlockSpec((1, _B, C_S),
                         lambda i: (i // nb_half, i % nb_half, 0)),
            pl.BlockSpec((_B, 1), lambda i: (i, 0)),
            pl.BlockSpec((C_S, C_S), lambda i: (0, 0)),
            pl.BlockSpec((1, C_S), lambda i: (0, 0)),
            pl.BlockSpec((8, C_S), lambda i: (0, 0)),
            pl.BlockSpec((1, 8), lambda i: (0, 0)),
        ],
        out_specs=pl.BlockSpec((_B, 8), lambda i: (i, 0)),
        out_shape=jax.ShapeDtypeStruct((NC * SEG_HALF, 8), jnp.float32),
    )(sums3, cnts, W1, b1r, W2p, b2p)

    return out8[:NUM_SEGMENTS, :6]
